# baseline (speedup 1.0000x reference)
import functools

import jax
import jax.numpy as jnp
from jax import lax
from jax.experimental import pallas as pl
from jax.experimental.pallas import tpu as pltpu
from jax.experimental.pallas import tpu_sc as plsc

B = 16384
NF = 26
VOCAB = 100000
H = 64

NC = 2
NS = 16
NW = NC * NS

CH = B // NW
H_WORDS = CH * H

VCH = 1024
NJ = (VOCAB + VCH - 1) // VCH
VPAD = NJ * VCH
NROWS = NF * VPAD


def _tc_preweight_body(tv_ref, w_ref, b_ref, o_ref):
    dims = (((0,), (1,)), ((), ()))
    d0 = lax.dot_general(tv_ref[0, :, : VCH // 2], w_ref[...], dims,
                         preferred_element_type=jnp.float32,
                         precision=lax.Precision.HIGHEST)
    d1 = lax.dot_general(tv_ref[0, :, VCH // 2:], w_ref[...], dims,
                         preferred_element_type=jnp.float32,
                         precision=lax.Precision.HIGHEST)
    o_ref[0] = jnp.concatenate([d0, d1], axis=1) + b_ref[...]


def _tc_preweight(tv, w, b2d):
    return pl.pallas_call(
        _tc_preweight_body,
        grid=(NF, NJ),
        in_specs=[
            pl.BlockSpec((1, H, VCH), lambda f, j: (f, 0, j)),
            pl.BlockSpec((H, H), lambda f, j: (0, 0)),
            pl.BlockSpec((1, 2 * H), lambda f, j: (0, 0)),
        ],
        out_specs=pl.BlockSpec((1, VCH // 2, 2 * H), lambda f, j: (f, j, 0)),
        out_shape=jax.ShapeDtypeStruct((NF, VPAD // 2, 2 * H), jnp.float32),
    )(tv, w, b2d)


def _sc_gather_sum_build():
    mesh = plsc.VectorSubcoreMesh(core_axis_name="c", subcore_axis_name="s")

    @functools.partial(
        pl.kernel,
        out_type=jax.ShapeDtypeStruct((B * H,), jnp.float32),
        mesh=mesh,
        compiler_params=pltpu.CompilerParams(use_tc_tiling_on_sc=False),
        scratch_types=[
            pltpu.VMEM((NF, CH), jnp.int32),
            pltpu.VMEM((CH, H), jnp.float32),
            pltpu.VMEM((CH, H), jnp.float32),
            pltpu.VMEM((H_WORDS,), jnp.float32),
            pltpu.SemaphoreType.DMA,
            pltpu.SemaphoreType.DMA,
        ],
    )
    def sc_gather_sum(u_hbm, idx_hbm, out_hbm, idx_v, buf0, buf1, h_v,
                      sem0, sem1):
        wid = lax.axis_index("s") * NC + lax.axis_index("c")
        base = wid * CH

        pltpu.sync_copy(idx_hbm.at[:, pl.ds(base, CH)], idx_v)

        def start(f, buf, sem):
            pltpu.async_copy(u_hbm.at[idx_v.at[f]], buf, sem)

        def wait(buf, sem):
            pltpu.make_async_copy(u_hbm.at[pl.ds(0, CH)], buf, sem).wait()

        zeros = jnp.zeros((16,), jnp.float32)

        def zero_body(i, carry):
            for k in range(8):
                h_v[pl.ds(i * 128 + k * 16, 16)] = zeros
            return carry

        lax.fori_loop(0, H_WORDS // 128, zero_body, 0)

        def accumulate(buf):
            def acc_body(i, carry):
                for k in range(8):
                    r = i * 8 + k
                    for c in range(H // 16):
                        plsc.addupdate(h_v.at[pl.ds(r * H + c * 16, 16)],
                                       buf[r, pl.ds(c * 16, 16)])
                return carry
            lax.fori_loop(0, CH // 8, acc_body, 0)

        start(0, buf0, sem0)
        start(1, buf1, sem1)

        def body(t, carry):
            f = t * 2
            wait(buf0, sem0)
            accumulate(buf0)
            start(f + 2, buf0, sem0)
            wait(buf1, sem1)
            accumulate(buf1)
            start(f + 3, buf1, sem1)
            return carry

        lax.fori_loop(0, NF // 2 - 1, body, 0)

        wait(buf0, sem0)
        accumulate(buf0)
        wait(buf1, sem1)
        accumulate(buf1)

        def relu_body(i, carry):
            for k in range(8):
                off = i * 128 + k * 16
                h_v[pl.ds(off, 16)] = jnp.maximum(h_v[pl.ds(off, 16)], 0.0)
            return carry

        lax.fori_loop(0, H_WORDS // 128, relu_body, 0)

        pltpu.sync_copy(h_v, out_hbm.at[pl.ds(wid * H_WORDS, H_WORDS)])

    return sc_gather_sum


_sc_gather_sum = _sc_gather_sum_build()


def kernel(x, tables, fc_w, fc_b):
    tv = jnp.transpose(tables, (0, 2, 1))
    bb = (jnp.concatenate([fc_b, fc_b]) * (1.0 / NF)).reshape(1, 2 * H)
    u = _tc_preweight(tv, fc_w, bb)
    u2 = u.reshape(NROWS, H)
    v = jnp.transpose(x.astype(jnp.int32))
    offs = (jnp.arange(NF, dtype=jnp.int32) * VPAD)[:, None]
    idx_t = (offs + (v & ~(VCH - 1)) + 2 * (v & (VCH // 2 - 1))
             + ((v >> 9) & 1))
    return _sc_gather_sum(u2, idx_t).reshape(B, H)


# speedup vs baseline: 1.9145x; 1.9145x over previous
import functools

import jax
import jax.numpy as jnp
from jax import lax
from jax.experimental import pallas as pl
from jax.experimental.pallas import tpu as pltpu
from jax.experimental.pallas import tpu_sc as plsc

B = 16384
NF = 26
VOCAB = 100000
H = 64

NC = 2
NS = 16
NW = NC * NS

CH = B // NW
H_WORDS = CH * H

VCH = 2048
NJ = (VOCAB + VCH - 1) // VCH
VPAD = NJ * VCH
NROWS = NF * VPAD


def _tc_preweight_body(tv_ref, w0_ref, w1_ref, b_ref, o_ref):
    dims = (((0,), (1,)), ((), ()))
    d0 = lax.dot_general(tv_ref[0, :, : VCH // 2], w0_ref[...], dims,
                         preferred_element_type=jnp.float32,
                         precision=lax.Precision.DEFAULT)
    d1 = lax.dot_general(tv_ref[0, :, VCH // 2:], w1_ref[...], dims,
                         preferred_element_type=jnp.float32,
                         precision=lax.Precision.DEFAULT)
    o_ref[0] = d0 + d1 + b_ref[...]


def _tc_preweight(tv, w0, w1, b2d):
    return pl.pallas_call(
        _tc_preweight_body,
        grid=(NF, NJ),
        in_specs=[
            pl.BlockSpec((1, H, VCH), lambda f, j: (f, 0, j)),
            pl.BlockSpec((2 * H, H), lambda f, j: (0, 0)),
            pl.BlockSpec((2 * H, H), lambda f, j: (0, 0)),
            pl.BlockSpec((1, 2 * H), lambda f, j: (0, 0)),
        ],
        out_specs=pl.BlockSpec((1, VCH // 2, 2 * H), lambda f, j: (f, j, 0)),
        out_shape=jax.ShapeDtypeStruct((NF, VPAD // 2, 2 * H), jnp.float32),
    )(tv, w0, w1, b2d)


def _sc_gather_sum_build():
    mesh = plsc.VectorSubcoreMesh(core_axis_name="c", subcore_axis_name="s")

    @functools.partial(
        pl.kernel,
        out_type=jax.ShapeDtypeStruct((B * H,), jnp.float32),
        mesh=mesh,
        compiler_params=pltpu.CompilerParams(use_tc_tiling_on_sc=False),
        scratch_types=[
            pltpu.VMEM((NF, CH), jnp.int32),
            pltpu.VMEM((CH, H), jnp.float32),
            pltpu.VMEM((CH, H), jnp.float32),
            pltpu.VMEM((H_WORDS,), jnp.float32),
            pltpu.SemaphoreType.DMA,
            pltpu.SemaphoreType.DMA,
        ],
    )
    def sc_gather_sum(u_hbm, idx_hbm, out_hbm, idx_v, buf0, buf1, h_v,
                      sem0, sem1):
        wid = lax.axis_index("s") * NC + lax.axis_index("c")
        base = wid * CH

        pltpu.sync_copy(idx_hbm.at[:, pl.ds(base, CH)], idx_v)

        def start(f, buf, sem):
            pltpu.async_copy(u_hbm.at[idx_v.at[f]], buf, sem)

        def wait(buf, sem):
            pltpu.make_async_copy(u_hbm.at[pl.ds(0, CH)], buf, sem).wait()

        zeros = jnp.zeros((16,), jnp.float32)

        def zero_body(i, carry):
            for k in range(8):
                h_v[pl.ds(i * 128 + k * 16, 16)] = zeros
            return carry

        lax.fori_loop(0, H_WORDS // 128, zero_body, 0)

        def accumulate(buf):
            def acc_body(i, carry):
                for k in range(8):
                    r = i * 8 + k
                    for c in range(H // 16):
                        plsc.addupdate(h_v.at[pl.ds(r * H + c * 16, 16)],
                                       buf[r, pl.ds(c * 16, 16)])
                return carry
            lax.fori_loop(0, CH // 8, acc_body, 0)

        start(0, buf0, sem0)
        start(1, buf1, sem1)

        def body(t, carry):
            f = t * 2
            wait(buf0, sem0)
            accumulate(buf0)
            start(f + 2, buf0, sem0)
            wait(buf1, sem1)
            accumulate(buf1)
            start(f + 3, buf1, sem1)
            return carry

        lax.fori_loop(0, NF // 2 - 1, body, 0)

        wait(buf0, sem0)
        accumulate(buf0)
        wait(buf1, sem1)
        accumulate(buf1)

        def relu_body(i, carry):
            for k in range(8):
                off = i * 128 + k * 16
                h_v[pl.ds(off, 16)] = jnp.maximum(h_v[pl.ds(off, 16)], 0.0)
            return carry

        lax.fori_loop(0, H_WORDS // 128, relu_body, 0)

        pltpu.sync_copy(h_v, out_hbm.at[pl.ds(wid * H_WORDS, H_WORDS)])

    return sc_gather_sum


_sc_gather_sum = _sc_gather_sum_build()


def kernel(x, tables, fc_w, fc_b):
    tv = jnp.transpose(tables, (0, 2, 1))
    bb = (jnp.concatenate([fc_b, fc_b]) * (1.0 / NF)).reshape(1, 2 * H)
    zw = jnp.zeros((H, H), jnp.float32)
    w0 = jnp.concatenate([fc_w, zw], axis=0)
    w1 = jnp.concatenate([zw, fc_w], axis=0)
    u = _tc_preweight(tv, w0, w1, bb)
    u2 = u.reshape(NROWS, H)
    v = jnp.transpose(x.astype(jnp.int32))
    offs = (jnp.arange(NF, dtype=jnp.int32) * VPAD)[:, None]
    idx_t = (offs + (v & ~(VCH - 1)) + 2 * (v & (VCH // 2 - 1))
             + ((v // (VCH // 2)) & 1))
    return _sc_gather_sum(u2, idx_t).reshape(B, H)


# speedup vs baseline: 2.6306x; 1.3741x over previous
import functools

import jax
import jax.numpy as jnp
from jax import lax
from jax.experimental import pallas as pl
from jax.experimental.pallas import tpu as pltpu
from jax.experimental.pallas import tpu_sc as plsc

B = 16384
NF = 26
VOCAB = 100000
H = 64

NC = 2
NS = 16
NW = NC * NS

CH = B // NW
H_WORDS = CH * H

VCH = 4096
NJ = (VOCAB + VCH - 1) // VCH
VPAD = NJ * VCH
NROWS = NF * VPAD


def _tc_preweight_body(tv_ref, w0_ref, w1_ref, b_ref, o_ref):
    dims = (((0,), (1,)), ((), ()))
    d0 = lax.dot_general(tv_ref[0, :, : VCH // 2], w0_ref[...], dims,
                         preferred_element_type=jnp.float32,
                         precision=lax.Precision.DEFAULT)
    d1 = lax.dot_general(tv_ref[0, :, VCH // 2:], w1_ref[...], dims,
                         preferred_element_type=jnp.float32,
                         precision=lax.Precision.DEFAULT)
    o_ref[0] = d0 + d1 + b_ref[...]


def _tc_preweight(tv, w0, w1, b2d):
    return pl.pallas_call(
        _tc_preweight_body,
        grid=(NF, NJ),
        in_specs=[
            pl.BlockSpec((1, H, VCH), lambda f, j: (f, 0, j)),
            pl.BlockSpec((2 * H, H), lambda f, j: (0, 0)),
            pl.BlockSpec((2 * H, H), lambda f, j: (0, 0)),
            pl.BlockSpec((1, 2 * H), lambda f, j: (0, 0)),
        ],
        out_specs=pl.BlockSpec((1, VCH // 2, 2 * H), lambda f, j: (f, j, 0)),
        out_shape=jax.ShapeDtypeStruct((NF, VPAD // 2, 2 * H), jnp.float32),
    )(tv, w0, w1, b2d)


def _sc_gather_sum_build():
    mesh = plsc.VectorSubcoreMesh(core_axis_name="c", subcore_axis_name="s")

    @functools.partial(
        pl.kernel,
        out_type=jax.ShapeDtypeStruct((B * H,), jnp.float32),
        mesh=mesh,
        compiler_params=pltpu.CompilerParams(use_tc_tiling_on_sc=False),
        scratch_types=[
            pltpu.VMEM((NF, CH), jnp.int32),
            pltpu.VMEM((CH, H), jnp.float32),
            pltpu.VMEM((CH, H), jnp.float32),
            pltpu.VMEM((H_WORDS,), jnp.float32),
            pltpu.SemaphoreType.DMA,
            pltpu.SemaphoreType.DMA,
        ],
    )
    def sc_gather_sum(u_hbm, idx_hbm, out_hbm, idx_v, buf0, buf1, h_v,
                      sem0, sem1):
        wid = lax.axis_index("s") * NC + lax.axis_index("c")
        base = wid * CH

        pltpu.sync_copy(idx_hbm.at[:, pl.ds(base, CH)], idx_v)

        def start(f, buf, sem):
            pltpu.async_copy(u_hbm.at[idx_v.at[f]], buf, sem)

        def wait(buf, sem):
            pltpu.make_async_copy(u_hbm.at[pl.ds(0, CH)], buf, sem).wait()

        zeros = jnp.zeros((16,), jnp.float32)

        def zero_body(i, carry):
            for k in range(8):
                h_v[pl.ds(i * 128 + k * 16, 16)] = zeros
            return carry

        lax.fori_loop(0, H_WORDS // 128, zero_body, 0)

        def accumulate(buf):
            def acc_body(i, carry):
                for k in range(8):
                    r = i * 8 + k
                    for c in range(H // 16):
                        plsc.addupdate(h_v.at[pl.ds(r * H + c * 16, 16)],
                                       buf[r, pl.ds(c * 16, 16)])
                return carry
            lax.fori_loop(0, CH // 8, acc_body, 0)

        start(0, buf0, sem0)
        start(1, buf1, sem1)

        def body(t, carry):
            f = t * 2
            wait(buf0, sem0)
            accumulate(buf0)
            start(f + 2, buf0, sem0)
            wait(buf1, sem1)
            accumulate(buf1)
            start(f + 3, buf1, sem1)
            return carry

        lax.fori_loop(0, NF // 2 - 1, body, 0)

        wait(buf0, sem0)
        accumulate(buf0)
        wait(buf1, sem1)
        accumulate(buf1)

        def relu_body(i, carry):
            for k in range(8):
                off = i * 128 + k * 16
                h_v[pl.ds(off, 16)] = jnp.maximum(h_v[pl.ds(off, 16)], 0.0)
            return carry

        lax.fori_loop(0, H_WORDS // 128, relu_body, 0)

        pltpu.sync_copy(h_v, out_hbm.at[pl.ds(wid * H_WORDS, H_WORDS)])

    return sc_gather_sum


_sc_gather_sum = _sc_gather_sum_build()


def kernel(x, tables, fc_w, fc_b):
    tv = jnp.transpose(tables, (0, 2, 1))
    bb = (jnp.concatenate([fc_b, fc_b]) * (1.0 / NF)).reshape(1, 2 * H)
    zw = jnp.zeros((H, H), jnp.float32)
    w0 = jnp.concatenate([fc_w, zw], axis=0)
    w1 = jnp.concatenate([zw, fc_w], axis=0)
    u = _tc_preweight(tv, w0, w1, bb)
    u2 = u.reshape(NROWS, H)
    v = jnp.transpose(x.astype(jnp.int32))
    offs = (jnp.arange(NF, dtype=jnp.int32) * VPAD)[:, None]
    idx_t = (offs + (v & ~(VCH - 1)) + 2 * (v & (VCH // 2 - 1))
             + ((v // (VCH // 2)) & 1))
    return _sc_gather_sum(u2, idx_t).reshape(B, H)


# speedup vs baseline: 3.2729x; 1.2442x over previous
import functools

import jax
import jax.numpy as jnp
from jax import lax
from jax.experimental import pallas as pl
from jax.experimental.pallas import tpu as pltpu
from jax.experimental.pallas import tpu_sc as plsc

B = 16384
NF = 26
VOCAB = 100000
H = 64

NC = 2
NS = 16
NW = NC * NS

CH = B // NW
H_WORDS = CH * H

VCH = 8192
NJ = (VOCAB + VCH - 1) // VCH
VPAD = NJ * VCH
NROWS = NF * VPAD


def _tc_preweight_body(tv_ref, w0_ref, w1_ref, b_ref, o_ref):
    dims = (((0,), (1,)), ((), ()))
    d0 = lax.dot_general(tv_ref[0, :, : VCH // 2], w0_ref[...], dims,
                         preferred_element_type=jnp.float32,
                         precision=lax.Precision.DEFAULT)
    d1 = lax.dot_general(tv_ref[0, :, VCH // 2:], w1_ref[...], dims,
                         preferred_element_type=jnp.float32,
                         precision=lax.Precision.DEFAULT)
    o_ref[0] = d0 + d1 + b_ref[...]


def _tc_preweight(tv, w0, w1, b2d):
    return pl.pallas_call(
        _tc_preweight_body,
        grid=(NF, NJ),
        in_specs=[
            pl.BlockSpec((1, H, VCH), lambda f, j: (f, 0, j)),
            pl.BlockSpec((2 * H, H), lambda f, j: (0, 0)),
            pl.BlockSpec((2 * H, H), lambda f, j: (0, 0)),
            pl.BlockSpec((1, 2 * H), lambda f, j: (0, 0)),
        ],
        out_specs=pl.BlockSpec((1, VCH // 2, 2 * H), lambda f, j: (f, j, 0)),
        out_shape=jax.ShapeDtypeStruct((NF, VPAD // 2, 2 * H), jnp.float32),
    )(tv, w0, w1, b2d)


def _sc_gather_sum_build():
    mesh = plsc.VectorSubcoreMesh(core_axis_name="c", subcore_axis_name="s")

    @functools.partial(
        pl.kernel,
        out_type=jax.ShapeDtypeStruct((B * H,), jnp.float32),
        mesh=mesh,
        compiler_params=pltpu.CompilerParams(use_tc_tiling_on_sc=False),
        scratch_types=[
            pltpu.VMEM((NF, CH), jnp.int32),
            pltpu.VMEM((CH, H), jnp.float32),
            pltpu.VMEM((CH, H), jnp.float32),
            pltpu.VMEM((H_WORDS,), jnp.float32),
            pltpu.SemaphoreType.DMA,
            pltpu.SemaphoreType.DMA,
        ],
    )
    def sc_gather_sum(u_hbm, idx_hbm, out_hbm, idx_v, buf0, buf1, h_v,
                      sem0, sem1):
        wid = lax.axis_index("s") * NC + lax.axis_index("c")
        base = wid * CH

        pltpu.sync_copy(idx_hbm.at[:, pl.ds(base, CH)], idx_v)

        def start(f, buf, sem):
            pltpu.async_copy(u_hbm.at[idx_v.at[f]], buf, sem)

        def wait(buf, sem):
            pltpu.make_async_copy(u_hbm.at[pl.ds(0, CH)], buf, sem).wait()

        zeros = jnp.zeros((16,), jnp.float32)

        def zero_body(i, carry):
            for k in range(8):
                h_v[pl.ds(i * 128 + k * 16, 16)] = zeros
            return carry

        lax.fori_loop(0, H_WORDS // 128, zero_body, 0)

        def accumulate(buf):
            def acc_body(i, carry):
                for k in range(8):
                    r = i * 8 + k
                    for c in range(H // 16):
                        plsc.addupdate(h_v.at[pl.ds(r * H + c * 16, 16)],
                                       buf[r, pl.ds(c * 16, 16)])
                return carry
            lax.fori_loop(0, CH // 8, acc_body, 0)

        start(0, buf0, sem0)
        start(1, buf1, sem1)

        def body(t, carry):
            f = t * 2
            wait(buf0, sem0)
            accumulate(buf0)
            start(f + 2, buf0, sem0)
            wait(buf1, sem1)
            accumulate(buf1)
            start(f + 3, buf1, sem1)
            return carry

        lax.fori_loop(0, NF // 2 - 1, body, 0)

        wait(buf0, sem0)
        accumulate(buf0)
        wait(buf1, sem1)
        accumulate(buf1)

        def relu_body(i, carry):
            for k in range(8):
                off = i * 128 + k * 16
                h_v[pl.ds(off, 16)] = jnp.maximum(h_v[pl.ds(off, 16)], 0.0)
            return carry

        lax.fori_loop(0, H_WORDS // 128, relu_body, 0)

        pltpu.sync_copy(h_v, out_hbm.at[pl.ds(wid * H_WORDS, H_WORDS)])

    return sc_gather_sum


_sc_gather_sum = _sc_gather_sum_build()


def kernel(x, tables, fc_w, fc_b):
    tv = jnp.transpose(tables, (0, 2, 1))
    bb = (jnp.concatenate([fc_b, fc_b]) * (1.0 / NF)).reshape(1, 2 * H)
    zw = jnp.zeros((H, H), jnp.float32)
    w0 = jnp.concatenate([fc_w, zw], axis=0)
    w1 = jnp.concatenate([zw, fc_w], axis=0)
    u = _tc_preweight(tv, w0, w1, bb)
    u2 = u.reshape(NROWS, H)
    v = jnp.transpose(x.astype(jnp.int32))
    offs = (jnp.arange(NF, dtype=jnp.int32) * VPAD)[:, None]
    idx_t = (offs + (v & ~(VCH - 1)) + 2 * (v & (VCH // 2 - 1))
             + ((v // (VCH // 2)) & 1))
    return _sc_gather_sum(u2, idx_t).reshape(B, H)


# speedup vs baseline: 3.7214x; 1.1370x over previous
import functools

import jax
import jax.numpy as jnp
from jax import lax
from jax.experimental import pallas as pl
from jax.experimental.pallas import tpu as pltpu
from jax.experimental.pallas import tpu_sc as plsc

B = 16384
NF = 26
VOCAB = 100000
H = 64

NC = 2
NS = 16
NW = NC * NS

CH = B // NW
H_WORDS = CH * H

VCH = 12800
NJ = (VOCAB + VCH - 1) // VCH
VPAD = NJ * VCH
NROWS = NF * VPAD


def _tc_preweight_body(tv_ref, w0_ref, w1_ref, b_ref, o_ref):
    dims = (((0,), (1,)), ((), ()))
    d0 = lax.dot_general(tv_ref[0, :, : VCH // 2], w0_ref[...], dims,
                         preferred_element_type=jnp.float32,
                         precision=lax.Precision.DEFAULT)
    d1 = lax.dot_general(tv_ref[0, :, VCH // 2:], w1_ref[...], dims,
                         preferred_element_type=jnp.float32,
                         precision=lax.Precision.DEFAULT)
    o_ref[0] = d0 + d1 + b_ref[...]


def _tc_preweight(tv, w0, w1, b2d):
    return pl.pallas_call(
        _tc_preweight_body,
        grid=(NF, NJ),
        in_specs=[
            pl.BlockSpec((1, H, VCH), lambda f, j: (f, 0, j)),
            pl.BlockSpec((2 * H, H), lambda f, j: (0, 0)),
            pl.BlockSpec((2 * H, H), lambda f, j: (0, 0)),
            pl.BlockSpec((1, 2 * H), lambda f, j: (0, 0)),
        ],
        out_specs=pl.BlockSpec((1, VCH // 2, 2 * H), lambda f, j: (f, j, 0)),
        out_shape=jax.ShapeDtypeStruct((NF, VPAD // 2, 2 * H), jnp.float32),
    )(tv, w0, w1, b2d)


def _sc_gather_sum_build():
    mesh = plsc.VectorSubcoreMesh(core_axis_name="c", subcore_axis_name="s")

    @functools.partial(
        pl.kernel,
        out_type=jax.ShapeDtypeStruct((B * H,), jnp.float32),
        mesh=mesh,
        compiler_params=pltpu.CompilerParams(use_tc_tiling_on_sc=False),
        scratch_types=[
            pltpu.VMEM((NF, CH), jnp.int32),
            pltpu.VMEM((CH, H), jnp.float32),
            pltpu.VMEM((CH, H), jnp.float32),
            pltpu.VMEM((H_WORDS,), jnp.float32),
            pltpu.SemaphoreType.DMA,
            pltpu.SemaphoreType.DMA,
        ],
    )
    def sc_gather_sum(u_hbm, idx_hbm, out_hbm, idx_v, buf0, buf1, h_v,
                      sem0, sem1):
        wid = lax.axis_index("s") * NC + lax.axis_index("c")
        base = wid * CH

        pltpu.sync_copy(idx_hbm.at[:, pl.ds(base, CH)], idx_v)

        def start(f, buf, sem):
            pltpu.async_copy(u_hbm.at[idx_v.at[f]], buf, sem)

        def wait(buf, sem):
            pltpu.make_async_copy(u_hbm.at[pl.ds(0, CH)], buf, sem).wait()

        zeros = jnp.zeros((16,), jnp.float32)

        def zero_body(i, carry):
            for k in range(8):
                h_v[pl.ds(i * 128 + k * 16, 16)] = zeros
            return carry

        lax.fori_loop(0, H_WORDS // 128, zero_body, 0)

        def accumulate(buf):
            def acc_body(i, carry):
                for k in range(8):
                    r = i * 8 + k
                    for c in range(H // 16):
                        plsc.addupdate(h_v.at[pl.ds(r * H + c * 16, 16)],
                                       buf[r, pl.ds(c * 16, 16)])
                return carry
            lax.fori_loop(0, CH // 8, acc_body, 0)

        start(0, buf0, sem0)
        start(1, buf1, sem1)

        def body(t, carry):
            f = t * 2
            wait(buf0, sem0)
            accumulate(buf0)
            start(f + 2, buf0, sem0)
            wait(buf1, sem1)
            accumulate(buf1)
            start(f + 3, buf1, sem1)
            return carry

        lax.fori_loop(0, NF // 2 - 1, body, 0)

        wait(buf0, sem0)
        accumulate(buf0)
        wait(buf1, sem1)
        accumulate(buf1)

        def relu_body(i, carry):
            for k in range(8):
                off = i * 128 + k * 16
                h_v[pl.ds(off, 16)] = jnp.maximum(h_v[pl.ds(off, 16)], 0.0)
            return carry

        lax.fori_loop(0, H_WORDS // 128, relu_body, 0)

        pltpu.sync_copy(h_v, out_hbm.at[pl.ds(wid * H_WORDS, H_WORDS)])

    return sc_gather_sum


_sc_gather_sum = _sc_gather_sum_build()


def kernel(x, tables, fc_w, fc_b):
    tv = jnp.transpose(tables, (0, 2, 1))
    bb = (jnp.concatenate([fc_b, fc_b]) * (1.0 / NF)).reshape(1, 2 * H)
    zw = jnp.zeros((H, H), jnp.float32)
    w0 = jnp.concatenate([fc_w, zw], axis=0)
    w1 = jnp.concatenate([zw, fc_w], axis=0)
    u = _tc_preweight(tv, w0, w1, bb)
    u2 = u.reshape(NROWS, H)
    v = jnp.transpose(x.astype(jnp.int32))
    offs = (jnp.arange(NF, dtype=jnp.int32) * VPAD)[:, None]
    idx_t = (offs + (v // VCH) * VCH + 2 * (v % (VCH // 2))
             + (v // (VCH // 2)) % 2)
    return _sc_gather_sum(u2, idx_t).reshape(B, H)


# speedup vs baseline: 4.1765x; 1.1223x over previous
import functools

import jax
import jax.numpy as jnp
from jax import lax
from jax.experimental import pallas as pl
from jax.experimental.pallas import tpu as pltpu
from jax.experimental.pallas import tpu_sc as plsc

B = 16384
NF = 26
VOCAB = 100000
H = 64

NC = 2
NS = 16
NW = NC * NS

CH = B // NW
H_WORDS = CH * H

VCH = 25600
NJ = (VOCAB + VCH - 1) // VCH
VPAD = NJ * VCH
NROWS = NF * VPAD


def _tc_preweight_body(tv_ref, w0_ref, w1_ref, b_ref, o_ref):
    dims = (((0,), (1,)), ((), ()))
    d0 = lax.dot_general(tv_ref[0, :, : VCH // 2], w0_ref[...], dims,
                         preferred_element_type=jnp.float32,
                         precision=lax.Precision.DEFAULT)
    d1 = lax.dot_general(tv_ref[0, :, VCH // 2:], w1_ref[...], dims,
                         preferred_element_type=jnp.float32,
                         precision=lax.Precision.DEFAULT)
    o_ref[0] = d0 + d1 + b_ref[...]


def _tc_preweight(tv, w0, w1, b2d):
    return pl.pallas_call(
        _tc_preweight_body,
        grid=(NF, NJ),
        in_specs=[
            pl.BlockSpec((1, H, VCH), lambda f, j: (f, 0, j)),
            pl.BlockSpec((2 * H, H), lambda f, j: (0, 0)),
            pl.BlockSpec((2 * H, H), lambda f, j: (0, 0)),
            pl.BlockSpec((1, 2 * H), lambda f, j: (0, 0)),
        ],
        out_specs=pl.BlockSpec((1, VCH // 2, 2 * H), lambda f, j: (f, j, 0)),
        out_shape=jax.ShapeDtypeStruct((NF, VPAD // 2, 2 * H), jnp.float32),
    )(tv, w0, w1, b2d)


def _sc_gather_sum_build():
    mesh = plsc.VectorSubcoreMesh(core_axis_name="c", subcore_axis_name="s")

    @functools.partial(
        pl.kernel,
        out_type=jax.ShapeDtypeStruct((B * H,), jnp.float32),
        mesh=mesh,
        compiler_params=pltpu.CompilerParams(use_tc_tiling_on_sc=False),
        scratch_types=[
            pltpu.VMEM((NF, CH), jnp.int32),
            pltpu.VMEM((CH, H), jnp.float32),
            pltpu.VMEM((CH, H), jnp.float32),
            pltpu.VMEM((H_WORDS,), jnp.float32),
            pltpu.SemaphoreType.DMA,
            pltpu.SemaphoreType.DMA,
        ],
    )
    def sc_gather_sum(u_hbm, idx_hbm, out_hbm, idx_v, buf0, buf1, h_v,
                      sem0, sem1):
        wid = lax.axis_index("s") * NC + lax.axis_index("c")
        base = wid * CH

        pltpu.sync_copy(idx_hbm.at[:, pl.ds(base, CH)], idx_v)

        def start(f, buf, sem):
            pltpu.async_copy(u_hbm.at[idx_v.at[f]], buf, sem)

        def wait(buf, sem):
            pltpu.make_async_copy(u_hbm.at[pl.ds(0, CH)], buf, sem).wait()

        zeros = jnp.zeros((16,), jnp.float32)

        def zero_body(i, carry):
            for k in range(8):
                h_v[pl.ds(i * 128 + k * 16, 16)] = zeros
            return carry

        lax.fori_loop(0, H_WORDS // 128, zero_body, 0)

        def accumulate(buf):
            def acc_body(i, carry):
                for k in range(8):
                    r = i * 8 + k
                    for c in range(H // 16):
                        plsc.addupdate(h_v.at[pl.ds(r * H + c * 16, 16)],
                                       buf[r, pl.ds(c * 16, 16)])
                return carry
            lax.fori_loop(0, CH // 8, acc_body, 0)

        start(0, buf0, sem0)
        start(1, buf1, sem1)

        def body(t, carry):
            f = t * 2
            wait(buf0, sem0)
            accumulate(buf0)
            start(f + 2, buf0, sem0)
            wait(buf1, sem1)
            accumulate(buf1)
            start(f + 3, buf1, sem1)
            return carry

        lax.fori_loop(0, NF // 2 - 1, body, 0)

        wait(buf0, sem0)
        accumulate(buf0)
        wait(buf1, sem1)
        accumulate(buf1)

        def relu_body(i, carry):
            for k in range(8):
                off = i * 128 + k * 16
                h_v[pl.ds(off, 16)] = jnp.maximum(h_v[pl.ds(off, 16)], 0.0)
            return carry

        lax.fori_loop(0, H_WORDS // 128, relu_body, 0)

        pltpu.sync_copy(h_v, out_hbm.at[pl.ds(wid * H_WORDS, H_WORDS)])

    return sc_gather_sum


_sc_gather_sum = _sc_gather_sum_build()


def kernel(x, tables, fc_w, fc_b):
    tv = jnp.transpose(tables, (0, 2, 1))
    bb = (jnp.concatenate([fc_b, fc_b]) * (1.0 / NF)).reshape(1, 2 * H)
    zw = jnp.zeros((H, H), jnp.float32)
    w0 = jnp.concatenate([fc_w, zw], axis=0)
    w1 = jnp.concatenate([zw, fc_w], axis=0)
    u = _tc_preweight(tv, w0, w1, bb)
    u2 = u.reshape(NROWS, H)
    v = jnp.transpose(x.astype(jnp.int32))
    offs = (jnp.arange(NF, dtype=jnp.int32) * VPAD)[:, None]
    idx_t = (offs + (v // VCH) * VCH + 2 * (v % (VCH // 2))
             + (v // (VCH // 2)) % 2)
    return _sc_gather_sum(u2, idx_t).reshape(B, H)
